# baseline (speedup 1.0000x reference)
import jax
import jax.numpy as jnp
from jax import lax
from jax.experimental import pallas as pl

N = 10000
E = 320000
HIDDEN = 128
NUM_LAYERS = 8
MLP_LAYERS = 2
G = 64
EPS = 1e-5
OUT_DIM = 64


def _tc_readout_body(h_ref, gw_ref, gb_ref, aw_ref, ab_ref,
                     bcol_ref, brow_ref, out_ref):
    hn = h_ref[...]
    gate = jnp.dot(hn, gw_ref[...],
                   preferred_element_type=jnp.float32) + gb_ref[...]
    bcol = bcol_ref[...]
    seg = lax.broadcasted_iota(jnp.int32, (N, G), 1)
    mask = bcol == seg
    gmax = jnp.max(jnp.where(mask, gate, -1e30), axis=0, keepdims=True)
    gmax_b = jnp.sum(jnp.where(mask, gmax, 0.0), axis=1, keepdims=True)
    e = jnp.exp(gate - gmax_b)
    denom = jnp.sum(jnp.where(mask, e, 0.0), axis=0, keepdims=True)
    denom_b = jnp.sum(jnp.where(mask, denom, 0.0), axis=1, keepdims=True)
    alpha = e / (denom_b + 1e-16)
    v = jnp.dot(hn, aw_ref[...], preferred_element_type=jnp.float32) + ab_ref[...]
    av = alpha * v
    brow = brow_ref[...]
    segt = lax.broadcasted_iota(jnp.int32, (G, N), 0)
    maskt = (segt == brow).astype(jnp.float32)
    out_ref[...] = jnp.dot(maskt, av, preferred_element_type=jnp.float32,
                           precision=lax.Precision.HIGHEST)


_tc_readout = pl.pallas_call(
    _tc_readout_body,
    out_shape=jax.ShapeDtypeStruct((G, HIDDEN), jnp.float32),
)


def _tc_head_body(c_ref, w1_ref, b1_ref, w2_ref, b2_ref, o_ref):
    hid = jnp.dot(c_ref[...], w1_ref[...], preferred_element_type=jnp.float32)
    hid = jnp.maximum(hid + b1_ref[...], 0.0)
    o_ref[...] = jnp.dot(hid, w2_ref[...],
                         preferred_element_type=jnp.float32) + b2_ref[...]


_tc_head = pl.pallas_call(
    _tc_head_body,
    out_shape=jax.ShapeDtypeStruct((G, OUT_DIM), jnp.float32),
)


def kernel(x, edge_weight, W_conv, b_conv, bn_gamma, bn_beta, gate_w, gate_b,
           att_w, att_b, pred_w1, pred_b1, pred_w2, pred_b2, edge_index, batch):
    src = edge_index[0]
    dst = edge_index[1]
    h = jnp.pad(x.reshape(-1, 1), ((0, 0), (0, HIDDEN - 1)))
    w = edge_weight.reshape(-1, 1)
    bcol = batch.reshape(N, 1).astype(jnp.int32)
    brow = batch.reshape(1, N).astype(jnp.int32)
    gb = gate_b.reshape(1, 1)
    ab_row = att_b.reshape(1, HIDDEN)
    outs = []
    for i in range(NUM_LAYERS):
        msg = h[src] * w
        agg = jax.ops.segment_sum(msg, dst, num_segments=N)
        m = h + agg
        for l in range(MLP_LAYERS):
            m = m @ W_conv[i, l] + b_conv[i, l]
            if l < MLP_LAYERS - 1:
                m = jax.nn.relu(m)
        h = jax.nn.relu(m)
        mean = jnp.mean(h, axis=0)
        var = jnp.mean((h - mean) ** 2, axis=0)
        h = (h - mean) / jnp.sqrt(var + EPS) * bn_gamma[i] + bn_beta[i]
        outs.append(_tc_readout(h, gate_w, gb, att_w, ab_row, bcol, brow))
    cat_h = jnp.concatenate(outs, axis=1)
    return _tc_head(cat_h, pred_w1, pred_b1.reshape(1, OUT_DIM),
                    pred_w2, pred_b2.reshape(1, OUT_DIM))
